# Initial kernel scaffold; baseline (speedup 1.0000x reference)
import jax
import jax.numpy as jnp
from jax.experimental import pallas as pl

N = 10000
E = 320000
D = 128
H = 4
C = 32


def _copy_kernel(x_ref, o_ref):
    o_ref[...] = x_ref[...]


def _conv(x, src, dst, edge_attr, W_src, W_edge, att, bias):
    n = x.shape[0]
    xs = (x @ W_src).reshape(n, H, C)
    loops = jnp.arange(n)
    s = jnp.concatenate([src, loops])
    d = jnp.concatenate([dst, loops])
    ea = jnp.concatenate([edge_attr, jnp.zeros((n, edge_attr.shape[1]), edge_attr.dtype)], axis=0)
    x_i = xs[d]
    x_j = xs[s]
    ea_p = (ea @ W_edge).reshape(-1, H, C)
    a = jnp.concatenate([x_i, x_j, ea_p], axis=2)
    a = jnp.einsum('hc,ehc->eh', att[:, 0, :], a)
    a = jnp.tanh(a)
    sgn = jnp.sign(a)
    aa = jnp.abs(a)
    m = jax.ops.segment_max(aa, d, num_segments=n)
    m = jax.lax.stop_gradient(jnp.where(jnp.isfinite(m), m, 0.0))
    ex = jnp.exp(aa - m[d])
    ssum = jax.ops.segment_sum(ex, d, num_segments=n)
    alpha = sgn * (ex / (ssum[d] + 1e-16))
    msg = alpha[..., None] * x_j
    out = jax.ops.segment_sum(msg, d, num_segments=n)
    return out.reshape(n, H * C) + bias


def kernel(entity_ids, edge_index, edge_type, edge_weight, entity_table, W_proj, b_proj, rel_emb, W_ew, b_ew, W_src0, W_edge0, att0, bias0, W_src1, W_edge1, att1, bias1):
    x = entity_table[entity_ids]
    x = x @ W_proj + b_proj
    edge_attr = rel_emb[edge_type] + edge_weight[:, None] @ W_ew + b_ew
    src, dst = edge_index[0], edge_index[1]
    h = _conv(x, src, dst, edge_attr, W_src0, W_edge0, att0, bias0)
    h = jax.nn.relu(h)
    h = _conv(h, src, dst, edge_attr, W_src1, W_edge1, att1, bias1)
    h = pl.pallas_call(
        _copy_kernel,
        out_shape=jax.ShapeDtypeStruct(h.shape, h.dtype),
    )(h)
    return h


# speedup vs baseline: 28.9644x; 28.9644x over previous
import functools

import jax
import jax.numpy as jnp
from jax import lax
from jax.experimental import pallas as pl
from jax.experimental.pallas import tpu as pltpu
from jax.experimental.pallas import tpu_sc as plsc

N = 10000
E = 320000
D = 128
H = 4
C = 32
R = 16

NCORES = 2
NSUB = 16
NTILES = NCORES * NSUB
PER_TILE = E // NTILES
EPC = 64
NCHUNK = PER_TILE // EPC
TAIL = PER_TILE - NCHUNK * EPC
NPAD = 10240
ROWS_PER_SUB = NPAD // NSUB
ROWS2 = NPAD // 32

_SC_MESH = plsc.VectorSubcoreMesh(core_axis_name="c", subcore_axis_name="s")
_SC_PARAMS = pltpu.CompilerParams(needs_layout_passes=False)


def _sc_edge_body(xs_hbm, adas_hbm, src_hbm, dst_hbm, base_hbm, zeros_hbm,
                  out_hbm, out2_hbm, acc, acc2, adas_sh,
                  src_v, dst_v, row2_v, rowd_v, rows_v_idx,
                  srct_v, dstt_v, row2t_v, rowdt_v, rowst_v,
                  base_v, rows_v, exrow_v, adrow_v, asrow_v, w_v,
                  semx, sema, semb):
    c = lax.axis_index("c")
    s = lax.axis_index("s")
    wid = s * NCORES + c
    tile_base = wid * PER_TILE
    srow = pl.multiple_of(s * ROWS_PER_SUB, 8)

    @pl.when(s < 8)
    def _stage_adas():
        pltpu.sync_copy(adas_hbm.at[pl.ds(pl.multiple_of(s * 80, 8), 80)],
                        adas_sh.at[pl.ds(pl.multiple_of(s * 80, 8), 80)])

    pltpu.sync_copy(zeros_hbm, acc.at[pl.ds(srow, ROWS_PER_SUB)])

    @pl.when(s < ROWS2 // 32)
    def _zero_acc2():
        pltpu.sync_copy(zeros_hbm.at[pl.ds(0, 32)],
                        acc2.at[pl.ds(pl.multiple_of(s * 32, 8), 32)])

    zero16 = jnp.zeros((16,), jnp.float32)

    @pl.loop(0, EPC)
    def _zero_exrow(e):
        for k in range(8):
            exrow_v[e, pl.ds(16 * k, 16)] = zero16

    plsc.subcore_barrier()

    iota16 = lax.iota(jnp.int32, 16)

    def do_chunk(off, ne, srcr, dstr, row2r, rowdr, rowsr):
        pltpu.sync_copy(src_hbm.at[pl.ds(off, ne)], srcr)
        pltpu.sync_copy(dst_hbm.at[pl.ds(off, ne)], dstr)
        for h in range(H):
            pltpu.sync_copy(base_hbm.at[pl.ds(pl.multiple_of(h * E + off, 8),
                                              ne)],
                            base_v.at[h, pl.ds(0, ne)])
        for j in range(ne // 16):
            srcv = srcr[pl.ds(16 * j, 16)]
            dstv = dstr[pl.ds(16 * j, 16)]
            rowdr[pl.ds(16 * j, 16)] = dstv >> 4
            rowsr[pl.ds(16 * j, 16)] = srcv >> 4
            row2r[pl.ds(16 * j, 16)] = dstv >> 5
        dx = pltpu.async_copy(xs_hbm.at[srcr], rows_v.at[pl.ds(0, ne)], semx)
        da = pltpu.async_copy(adas_sh.at[rowdr], adrow_v.at[pl.ds(0, ne)],
                              sema)
        db = pltpu.async_copy(adas_sh.at[rowsr], asrow_v.at[pl.ds(0, ne)],
                              semb)
        da.wait()
        db.wait()

        for j in range(ne // 16):
            eids = iota16 + 16 * j
            srcv = srcr[pl.ds(16 * j, 16)]
            dstv = dstr[pl.ds(16 * j, 16)]
            lane_ad = (dstv & 15) * 8
            lane_as = (srcv & 15) * 8 + 4
            lane_ex = (dstv & 31) * 4
            for h in range(H):
                lg = (plsc.load_gather(adrow_v, [eids, lane_ad + h])
                      + plsc.load_gather(asrow_v, [eids, lane_as + h])
                      + base_v[h, pl.ds(16 * j, 16)])
                z = jnp.exp(-2.0 * jnp.abs(lg))
                aa = (1.0 - z) / (1.0 + z)
                exv = jnp.exp(aa)
                w_v[pl.ds(h * EPC + 16 * j, 16)] = jnp.sign(lg) * exv
                plsc.store_scatter(exrow_v, [eids, lane_ex + h], exv)

        dx.wait()

        @pl.loop(0, ne)
        def _mul(e):
            for h in range(H):
                wvec = plsc.load_gather(
                    w_v, [jnp.full((16,), h * EPC, jnp.int32) + e])
                rows_v[e, pl.ds(32 * h, 16)] = (
                    rows_v[e, pl.ds(32 * h, 16)] * wvec)
                rows_v[e, pl.ds(32 * h + 16, 16)] = (
                    rows_v[e, pl.ds(32 * h + 16, 16)] * wvec)

        pltpu.sync_copy(rows_v.at[pl.ds(0, ne)], acc.at[dstr], add=True)
        pltpu.sync_copy(exrow_v.at[pl.ds(0, ne)], acc2.at[row2r], add=True)

        for j in range(ne // 16):
            eids = iota16 + 16 * j
            dstv = dstr[pl.ds(16 * j, 16)]
            lane_ex = (dstv & 31) * 4
            for h in range(H):
                plsc.store_scatter(exrow_v, [eids, lane_ex + h], zero16)

    @pl.loop(0, NCHUNK)
    def _chunk(i):
        off = pl.multiple_of(tile_base + i * EPC, 8)
        do_chunk(off, EPC, src_v, dst_v, row2_v, rowd_v, rows_v_idx)

    do_chunk(pl.multiple_of(tile_base + NCHUNK * EPC, 8), TAIL,
             srct_v, dstt_v, row2t_v, rowdt_v, rowst_v)

    plsc.subcore_barrier()
    pltpu.sync_copy(acc.at[pl.ds(srow, ROWS_PER_SUB)],
                    out_hbm.at[pl.ds(pl.multiple_of(c * NPAD + srow, 8),
                                     ROWS_PER_SUB)])

    @pl.when(s == 0)
    def _dump_acc2():
        pltpu.sync_copy(acc2,
                        out2_hbm.at[pl.ds(pl.multiple_of(c * ROWS2, 8),
                                          ROWS2)])


_sc_edge = functools.partial(
    pl.kernel,
    _sc_edge_body,
    out_type=(jax.ShapeDtypeStruct((NCORES * NPAD, D), jnp.float32),
              jax.ShapeDtypeStruct((NCORES * ROWS2, D), jnp.float32)),
    mesh=_SC_MESH,
    compiler_params=_SC_PARAMS,
    scratch_types=[
        pltpu.VMEM_SHARED((NPAD, D), jnp.float32),
        pltpu.VMEM_SHARED((ROWS2, D), jnp.float32),
        pltpu.VMEM_SHARED((640, D), jnp.float32),
        pltpu.VMEM((EPC,), jnp.int32),
        pltpu.VMEM((EPC,), jnp.int32),
        pltpu.VMEM((EPC,), jnp.int32),
        pltpu.VMEM((EPC,), jnp.int32),
        pltpu.VMEM((EPC,), jnp.int32),
        pltpu.VMEM((TAIL,), jnp.int32),
        pltpu.VMEM((TAIL,), jnp.int32),
        pltpu.VMEM((TAIL,), jnp.int32),
        pltpu.VMEM((TAIL,), jnp.int32),
        pltpu.VMEM((TAIL,), jnp.int32),
        pltpu.VMEM((H, EPC), jnp.float32),
        pltpu.VMEM((EPC, D), jnp.float32),
        pltpu.VMEM((EPC, D), jnp.float32),
        pltpu.VMEM((EPC, D), jnp.float32),
        pltpu.VMEM((EPC, D), jnp.float32),
        pltpu.VMEM((H * EPC,), jnp.float32),
        pltpu.SemaphoreType.DMA,
        pltpu.SemaphoreType.DMA,
        pltpu.SemaphoreType.DMA,
    ],
)()


def _node_prep_body(x_ref, wproj_ref, bproj_ref, wsrc_ref, a1_ref, a2_ref,
                    xs_ref, adas_ref):
    x1 = jnp.dot(x_ref[...], wproj_ref[...],
                 preferred_element_type=jnp.float32, precision=jax.lax.Precision.HIGHEST) + bproj_ref[...]
    xs = jnp.dot(x1, wsrc_ref[...], preferred_element_type=jnp.float32, precision=jax.lax.Precision.HIGHEST)
    xs_ref[...] = xs
    ad = jnp.dot(xs, a1_ref[...], preferred_element_type=jnp.float32, precision=jax.lax.Precision.HIGHEST)
    asrc = jnp.dot(xs, a2_ref[...], preferred_element_type=jnp.float32, precision=jax.lax.Precision.HIGHEST)
    adas_ref[...] = jnp.concatenate([ad, asrc], axis=1)


def _edge_prep_body(et_ref, ew_ref, relemb_ref, we0_ref, we1_ref, wew_ref,
                    bew_ref, a30_ref, a31_ref, b0_ref, b1_ref):
    et = et_ref[...]
    ew = ew_ref[...]
    for we_ref, a3_ref, out_ref in ((we0_ref, a30_ref, b0_ref),
                                    (we1_ref, a31_ref, b1_ref)):
        relproj = jnp.dot(relemb_ref[...], we_ref[...],
                          preferred_element_type=jnp.float32, precision=jax.lax.Precision.HIGHEST)
        rb = jnp.dot(relproj, a3_ref[...], preferred_element_type=jnp.float32, precision=jax.lax.Precision.HIGHEST)
        wb = jnp.dot(jnp.dot(wew_ref[...], we_ref[...],
                             preferred_element_type=jnp.float32, precision=jax.lax.Precision.HIGHEST),
                     a3_ref[...], preferred_element_type=jnp.float32, precision=jax.lax.Precision.HIGHEST)
        bb = jnp.dot(jnp.dot(bew_ref[...], we_ref[...],
                             preferred_element_type=jnp.float32, precision=jax.lax.Precision.HIGHEST),
                     a3_ref[...], preferred_element_type=jnp.float32, precision=jax.lax.Precision.HIGHEST)
        for h in range(H):
            acc = ew * wb[0, h] + bb[0, h]
            for r in range(R):
                acc = acc + jnp.where(et == r, rb[r, h], 0.0)
            out_ref[h] = acc


def _combine(acca, accb, s2a, s2b, adas, xs, bias, sel):
    num = acca[...] + accb[...]
    ssum = s2a[...] + s2b[...]
    ad = adas[:, 0:4]
    asrc = adas[:, 4:8]
    t = jnp.tanh(ad + asrc)
    ex = jnp.exp(jnp.abs(t))
    w = jnp.sign(t) * ex
    num = num + jnp.dot(w, sel[...], preferred_element_type=jnp.float32, precision=jax.lax.Precision.HIGHEST) * xs[...]
    den = jnp.dot(ssum + ex, sel[...], preferred_element_type=jnp.float32, precision=jax.lax.Precision.HIGHEST) + 1e-16
    return num / den + bias[...]


def _mid_body(acca, accb, s2a, s2b, adas, xs, bias, sel, wsrc1_ref, a11_ref,
              a21_ref, xs1_ref, adas1_ref):
    h1 = jnp.maximum(_combine(acca, accb, s2a, s2b, adas, xs, bias, sel), 0.0)
    xs1 = jnp.dot(h1, wsrc1_ref[...], preferred_element_type=jnp.float32, precision=jax.lax.Precision.HIGHEST)
    xs1_ref[...] = xs1
    ad = jnp.dot(xs1, a11_ref[...], preferred_element_type=jnp.float32, precision=jax.lax.Precision.HIGHEST)
    asrc = jnp.dot(xs1, a21_ref[...], preferred_element_type=jnp.float32, precision=jax.lax.Precision.HIGHEST)
    adas1_ref[...] = jnp.concatenate([ad, asrc], axis=1)


def _final_body(acca, accb, s2a, s2b, adas, xs, bias, sel, out_ref):
    out_ref[...] = _combine(acca, accb, s2a, s2b, adas, xs, bias, sel)


def _att_mats(att):
    eye = jnp.eye(H, dtype=jnp.float32)
    mats = []
    for k in range(3):
        a = att[:, 0, k * C:(k + 1) * C]
        mats.append(jnp.einsum('hc,hg->hcg', a, eye).reshape(H * C, H))
    return mats


def kernel(entity_ids, edge_index, edge_type, edge_weight, entity_table,
           W_proj, b_proj, rel_emb, W_ew, b_ew, W_src0, W_edge0, att0, bias0,
           W_src1, W_edge1, att1, bias1):
    f32 = jnp.float32
    x0 = entity_table[entity_ids].astype(f32)

    a10, a20, a30 = _att_mats(att0)
    a11, a21, a31 = _att_mats(att1)

    NB = 2000
    NGRID = N // NB

    def rows_spec(cols):
        return pl.BlockSpec((NB, cols), lambda i: (i, 0))

    def full_spec(shape):
        return pl.BlockSpec(shape, lambda i: tuple(0 for _ in shape))

    xs0, adas0 = pl.pallas_call(
        _node_prep_body,
        grid=(NGRID,),
        in_specs=[rows_spec(D), full_spec((D, D)), full_spec((1, D)),
                  full_spec((D, D)), full_spec((D, H)), full_spec((D, H))],
        out_specs=[rows_spec(D), rows_spec(2 * H)],
        out_shape=(jax.ShapeDtypeStruct((N, D), f32),
                   jax.ShapeDtypeStruct((N, 2 * H), f32)),
    )(x0, W_proj, b_proj.reshape(1, D), W_src0, a10, a20)

    EB = 2500
    et2 = edge_type.reshape(EB, 128).astype(jnp.int32)
    ew2 = edge_weight.reshape(EB, 128).astype(f32)
    base0, base1 = pl.pallas_call(
        _edge_prep_body,
        out_shape=(jax.ShapeDtypeStruct((H, EB, 128), f32),
                   jax.ShapeDtypeStruct((H, EB, 128), f32)),
    )(et2, ew2, rel_emb, W_edge0, W_edge1, W_ew, b_ew.reshape(1, D), a30, a31)
    base0 = base0.reshape(H, E)
    base1 = base1.reshape(H, E)

    src = edge_index[0].astype(jnp.int32)
    dst = edge_index[1].astype(jnp.int32)
    zeros = jnp.zeros((ROWS_PER_SUB, D), f32)
    sel = jnp.repeat(jnp.eye(H, dtype=f32), C, axis=1)
    base0 = base0.reshape(H * E)
    base1 = base1.reshape(H * E)

    def pack_adas(adas):
        padded = jnp.concatenate(
            [adas, jnp.zeros((NPAD - N, 2 * H), adas.dtype)], axis=0)
        return padded.reshape(NPAD // 16, 128)

    def unpack2(a2):
        pa = a2[:ROWS2].reshape(NPAD, H)[:N]
        pb = a2[ROWS2:].reshape(NPAD, H)[:N]
        return pa, pb

    acc0, accp0 = _sc_edge(xs0, pack_adas(adas0), src, dst, base0, zeros)
    s2a0, s2b0 = unpack2(accp0)

    combine_in_specs = [rows_spec(D), rows_spec(D), rows_spec(H),
                        rows_spec(H), rows_spec(2 * H), rows_spec(D),
                        full_spec((1, D)), full_spec((H, D))]

    xs1, adas1 = pl.pallas_call(
        _mid_body,
        grid=(NGRID,),
        in_specs=combine_in_specs + [full_spec((D, D)), full_spec((D, H)),
                                     full_spec((D, H))],
        out_specs=[rows_spec(D), rows_spec(2 * H)],
        out_shape=(jax.ShapeDtypeStruct((N, D), f32),
                   jax.ShapeDtypeStruct((N, 2 * H), f32)),
    )(acc0[:N], acc0[NPAD:NPAD + N], s2a0, s2b0, adas0, xs0,
      bias0.reshape(1, D), sel, W_src1, a11, a21)

    acc1, accp1 = _sc_edge(xs1, pack_adas(adas1), src, dst, base1, zeros)
    s2a1, s2b1 = unpack2(accp1)

    out = pl.pallas_call(
        _final_body,
        grid=(NGRID,),
        in_specs=combine_in_specs,
        out_specs=rows_spec(D),
        out_shape=jax.ShapeDtypeStruct((N, D), f32),
    )(acc1[:N], acc1[NPAD:NPAD + N], s2a1, s2b1, adas1, xs1,
      bias1.reshape(1, D), sel)
    return out
